# baseline (speedup 1.0000x reference)
import functools

import jax
import jax.numpy as jnp
from jax import lax
from jax.experimental import pallas as pl
from jax.experimental.pallas import tpu as pltpu


def _round_up(x: int, m: int) -> int:
    return ((x + m - 1) // m) * m


def _cdiv(a: int, b: int) -> int:
    return (a + b - 1) // b


def _unpool_kernel(idx_ref, h_ref, out_ref, hbf_ref):
    tile_n = out_ref.shape[0]
    m_pad = h_ref.shape[0]

    @pl.when(pl.program_id(0) == 0)
    def _():
        hbf_ref[...] = h_ref[...].astype(jnp.bfloat16)

    row0 = pl.program_id(0) * tile_n
    rows = lax.broadcasted_iota(jnp.int32, (tile_n, m_pad), 0) + row0
    onehot = (rows == idx_ref[...]).astype(jnp.bfloat16)

    out_ref[...] = jnp.dot(
        onehot, hbf_ref[...],
        preferred_element_type=jnp.float32,
    ).astype(out_ref.dtype)


@functools.partial(jax.jit, static_argnums=(0, 3))
def _unpool(node_nums: int, h: jax.Array, idx: jax.Array,
            tile_n: int = 1024) -> jax.Array:
    assert h.ndim == 2 and idx.ndim == 1 and idx.shape[0] == h.shape[0]
    m, d = h.shape

    if node_nums == 0 or d == 0 or m == 0:
        return jnp.zeros((node_nums, d), h.dtype)

    m_pad = _round_up(m, 128)
    h_in = h if m_pad == m else jnp.pad(h, ((0, m_pad - m), (0, 0)))
    if m_pad == m:
        idx_in = idx.astype(jnp.int32).reshape(1, m)
    else:
        idx_in = jnp.full((1, m_pad), -1, jnp.int32).at[0, :m].set(
            idx.astype(jnp.int32))

    tile_n_eff = min(tile_n, _round_up(node_nums, 8))
    grid_n = _cdiv(node_nums, tile_n_eff)

    cost = pl.CostEstimate(
        flops=2 * node_nums * m_pad * d,
        transcendentals=0,
        bytes_accessed=4 * m_pad * d + 4 * node_nums * d + 4 * m_pad,
    )

    out = pl.pallas_call(
        _unpool_kernel,
        out_shape=jax.ShapeDtypeStruct((node_nums, d), jnp.float32),
        grid=(grid_n,),
        in_specs=[
            pl.BlockSpec((1, m_pad), lambda i: (0, 0)),
            pl.BlockSpec((m_pad, d), lambda i: (0, 0)),
        ],
        out_specs=pl.BlockSpec((tile_n_eff, d), lambda i: (i, 0)),
        scratch_shapes=[pltpu.VMEM((m_pad, d), jnp.bfloat16)],
        compiler_params=pltpu.CompilerParams(
            dimension_semantics=("arbitrary",),
            vmem_limit_bytes=64 * 1024 * 1024,
        ),
        cost_estimate=cost,
    )(idx_in, h_in)
    return out.astype(h.dtype)


def kernel(h, idx):
    return _unpool(8192, h, idx)


# speedup vs baseline: 1.0059x; 1.0059x over previous
import functools

import jax
import jax.numpy as jnp
from jax import lax
from jax.experimental import pallas as pl
from jax.experimental.pallas import tpu as pltpu


def _round_up(x: int, m: int) -> int:
    return ((x + m - 1) // m) * m


def _cdiv(a: int, b: int) -> int:
    return (a + b - 1) // b


def _unpool_kernel(idx_ref, h_ref, out_ref):
    tile_n = out_ref.shape[0]
    m_pad = h_ref.shape[0]

    row0 = pl.program_id(0) * tile_n
    rows = lax.broadcasted_iota(jnp.int32, (tile_n, m_pad), 0) + row0
    onehot = (rows == idx_ref[...]).astype(jnp.float32)

    out_ref[...] = jnp.dot(
        onehot, h_ref[...],
        preferred_element_type=jnp.float32,
    ).astype(out_ref.dtype)


@functools.partial(jax.jit, static_argnums=(0, 3))
def _unpool(node_nums: int, h: jax.Array, idx: jax.Array,
            tile_n: int = 1024) -> jax.Array:
    assert h.ndim == 2 and idx.ndim == 1 and idx.shape[0] == h.shape[0]
    m, d = h.shape

    if node_nums == 0 or d == 0 or m == 0:
        return jnp.zeros((node_nums, d), h.dtype)

    m_pad = _round_up(m, 128)
    h_in = h if m_pad == m else jnp.pad(h, ((0, m_pad - m), (0, 0)))
    if m_pad == m:
        idx_in = idx.astype(jnp.int32).reshape(1, m)
    else:
        idx_in = jnp.full((1, m_pad), -1, jnp.int32).at[0, :m].set(
            idx.astype(jnp.int32))

    tile_n_eff = min(tile_n, _round_up(node_nums, 8))
    grid_n = _cdiv(node_nums, tile_n_eff)

    cost = pl.CostEstimate(
        flops=2 * node_nums * m_pad * d,
        transcendentals=0,
        bytes_accessed=4 * m_pad * d + 4 * node_nums * d + 4 * m_pad,
    )

    out = pl.pallas_call(
        _unpool_kernel,
        out_shape=jax.ShapeDtypeStruct((node_nums, d), jnp.float32),
        grid=(grid_n,),
        in_specs=[
            pl.BlockSpec((1, m_pad), lambda i: (0, 0)),
            pl.BlockSpec((m_pad, d), lambda i: (0, 0)),
        ],
        out_specs=pl.BlockSpec((tile_n_eff, d), lambda i: (i, 0)),
        compiler_params=pltpu.CompilerParams(
            dimension_semantics=("arbitrary",),
            vmem_limit_bytes=64 * 1024 * 1024,
        ),
        cost_estimate=cost,
    )(idx_in, h_in)
    return out.astype(h.dtype)


def kernel(h, idx):
    return _unpool(8192, h, idx)
